# baseline (speedup 1.0000x reference)
import jax
import jax.numpy as jnp
from jax.experimental import pallas as pl
from jax.experimental.pallas import tpu as pltpu


_K = 32


def _copy_body(hp_ref, qt_ref, hp_out, qt_out, hp_sem, qt_sems):
    rows = qt_ref.shape[0] // _K
    dmas = [
        pltpu.make_async_copy(
            qt_ref.at[pl.ds(k * rows, rows)],
            qt_out.at[pl.ds(k * rows, rows)],
            qt_sems.at[k],
        )
        for k in range(_K)
    ]
    hp_dma = pltpu.make_async_copy(hp_ref, hp_out, hp_sem)
    hp_dma.start()
    for d in dmas:
        d.start()
    hp_dma.wait()
    for d in dmas:
        d.wait()


def kernel(hip_pos, quat):
    hp = hip_pos.reshape(1024, 128 * 1 * 3)
    qt = quat.reshape(1024, 128 * 52 * 4)
    hp_o, qt_o = pl.pallas_call(
        _copy_body,
        in_specs=[
            pl.BlockSpec(memory_space=pl.ANY),
            pl.BlockSpec(memory_space=pl.ANY),
        ],
        out_specs=[
            pl.BlockSpec(memory_space=pl.ANY),
            pl.BlockSpec(memory_space=pl.ANY),
        ],
        out_shape=[
            jax.ShapeDtypeStruct(hp.shape, hp.dtype),
            jax.ShapeDtypeStruct(qt.shape, qt.dtype),
        ],
        scratch_shapes=[pltpu.SemaphoreType.DMA, pltpu.SemaphoreType.DMA((_K,))],
    )(hp, qt)
    return hp_o.reshape(hip_pos.shape), qt_o.reshape(quat.shape)


# speedup vs baseline: 5.9116x; 5.9116x over previous
import jax
import jax.numpy as jnp
from jax.experimental import pallas as pl
from jax.experimental.pallas import tpu as pltpu

_B = 1024
_HP_W = 128 * 1 * 3
_QT_W = 128 * 52 * 4
_G = 8
_N = _B // _G
_S = 32
_D = _S // 2


def _copy_body(hp_ref, qt_ref, hp_out, qt_out,
               qt_buf, hp_buf, qt_isem, qt_osem, hp_isem, hp_osem):
    hp_in = pltpu.make_async_copy(hp_ref, hp_buf, hp_isem)
    hp_in.start()

    in_dmas = [
        pltpu.make_async_copy(
            qt_ref.at[pl.ds(i * _G, _G)], qt_buf.at[i % _S], qt_isem.at[i % _S]
        )
        for i in range(_N)
    ]
    out_dmas = [
        pltpu.make_async_copy(
            qt_buf.at[i % _S], qt_out.at[pl.ds(i * _G, _G)], qt_osem.at[i % _S]
        )
        for i in range(_N)
    ]

    started = [False] * _N
    waited = [False] * _N

    def drain(j):
        in_dmas[j].wait()
        out_dmas[j].start()
        started[j] = True

    for i in range(_N):
        if i >= _S:
            j = i - _S
            if not started[j]:
                drain(j)
            out_dmas[j].wait()
            waited[j] = True
        in_dmas[i].start()
        j = i - _D
        if j >= 0 and not started[j]:
            drain(j)

    hp_in.wait()
    hp_o = pltpu.make_async_copy(hp_buf, hp_out, hp_osem)
    hp_o.start()

    for j in range(_N):
        if not started[j]:
            drain(j)
    for j in range(_N):
        if not waited[j]:
            out_dmas[j].wait()
    hp_o.wait()


def kernel(hip_pos, quat):
    hp = hip_pos.reshape(_B, _HP_W)
    qt = quat.reshape(_B, _QT_W)
    hp_o, qt_o = pl.pallas_call(
        _copy_body,
        in_specs=[
            pl.BlockSpec(memory_space=pl.ANY),
            pl.BlockSpec(memory_space=pl.ANY),
        ],
        out_specs=[
            pl.BlockSpec(memory_space=pl.ANY),
            pl.BlockSpec(memory_space=pl.ANY),
        ],
        out_shape=[
            jax.ShapeDtypeStruct(hp.shape, hp.dtype),
            jax.ShapeDtypeStruct(qt.shape, qt.dtype),
        ],
        scratch_shapes=[
            pltpu.VMEM((_S, _G, _QT_W), jnp.float32),
            pltpu.VMEM((_B, _HP_W), jnp.float32),
            pltpu.SemaphoreType.DMA((_S,)),
            pltpu.SemaphoreType.DMA((_S,)),
            pltpu.SemaphoreType.DMA,
            pltpu.SemaphoreType.DMA,
        ],
    )(hp, qt)
    return hp_o.reshape(hip_pos.shape), qt_o.reshape(quat.shape)
